# baseline (speedup 1.0000x reference)
import jax
import jax.numpy as jnp
from jax import lax
from jax.experimental import pallas as pl
from jax.experimental.pallas import tpu as pltpu
from jax.experimental.pallas import tpu_sc as plsc

B = 1024
S = 200
N = B * S
EMB = 64
POS_DIM = 32
OUT_D = EMB + POS_DIM
PAD_D = 128

NW = 32
NT = N // NW
C = 128
NBUF = 5
NB = NT // C


def _emb_body(widx_hbm, pidx_hbm, wtab_hbm, ptab_hbm, out_hbm,
              idx_w, idx_p, comb_v, sem_i, sem_w, sem_p, sem_o):
    wid = lax.axis_index("s") * 2 + lax.axis_index("c")
    base_t = wid * NT

    def base_of(g):
        return jnp.minimum(base_t + g * C, N - C)

    def idx_copies(g, b):
        base = base_of(g)
        return [
            pltpu.make_async_copy(widx_hbm.at[pl.ds(base, C)],
                                  idx_w.at[b], sem_i),
            pltpu.make_async_copy(pidx_hbm.at[pl.ds(base, C)],
                                  idx_p.at[b], sem_i),
        ]

    def word_gather(b):
        return pltpu.make_async_copy(wtab_hbm.at[idx_w.at[b]],
                                     comb_v.at[b], sem_w)

    def pos_add(b):
        return pltpu.async_copy(ptab_hbm.at[idx_p.at[b]], comb_v.at[b],
                                sem_p, add=True)

    def out_copy(g, b):
        return pltpu.make_async_copy(comb_v.at[b],
                                     out_hbm.at[pl.ds(base_of(g), C)],
                                     sem_o)

    def round_(k, first):
        gs = [k * NBUF + j for j in range(NBUF)]
        for j, g in enumerate(gs):
            for c in idx_copies(g, j):
                c.wait()
            if not first:
                out_copy(g - NBUF, j).wait()
            word_gather(j).start()
        adds = []
        for j, g in enumerate(gs):
            word_gather(j).wait()
            adds.append(pos_add(j))
        for j, g in enumerate(gs):
            adds[j].wait()
            out_copy(g, j).start()
            for c in idx_copies(g + NBUF, j):
                c.start()

    for j in range(NBUF):
        for c in idx_copies(j, j):
            c.start()
    round_(0, True)
    lax.fori_loop(1, NB // NBUF, lambda k, _: (round_(k, False), ())[1], ())
    for j in range(NBUF):
        out_copy(NB - NBUF + j, j).wait()
        for c in idx_copies(NB + j, j):
            c.wait()


@jax.jit
def _emb_call(widx, pidx, wtab_p, ptab_s):
    mesh = plsc.VectorSubcoreMesh(core_axis_name="c", subcore_axis_name="s")
    f = pl.kernel(
        _emb_body,
        out_type=jax.ShapeDtypeStruct((N, PAD_D), jnp.float32),
        mesh=mesh,
        scratch_types=[
            pltpu.VMEM((NBUF, C), jnp.int32),
            pltpu.VMEM((NBUF, C), jnp.int32),
            pltpu.VMEM((NBUF, C, PAD_D), jnp.float32),
            pltpu.SemaphoreType.DMA,
            pltpu.SemaphoreType.DMA,
            pltpu.SemaphoreType.DMA,
            pltpu.SemaphoreType.DMA,
        ],
    )
    return f(widx, pidx, wtab_p, ptab_s)


def kernel(word_inputs, pos_inputs, word_table, pos_table):
    widx = word_inputs.astype(jnp.int32).reshape(N)
    pidx = pos_inputs.astype(jnp.int32).reshape(N)
    wtab_p = jnp.pad(word_table, ((0, 0), (0, PAD_D - EMB)))
    ptab_s = jnp.pad(pos_table, ((0, 0), (EMB, PAD_D - EMB - POS_DIM)))
    out = _emb_call(widx, pidx, wtab_p, ptab_s)
    return out[:, :OUT_D].reshape(B, S, OUT_D)


# speedup vs baseline: 1.1861x; 1.1861x over previous
import jax
import jax.numpy as jnp
from jax import lax
from jax.experimental import pallas as pl
from jax.experimental.pallas import tpu as pltpu
from jax.experimental.pallas import tpu_sc as plsc

B = 1024
S = 200
N = B * S
EMB = 64
POS_DIM = 32
OUT_D = EMB + POS_DIM
PAD_D = 128

NW = 32
NT = N // NW
C = 128
NBUF = 5
NB = NT // C
L = 16


def _emb_body(widx_hbm, pidx_hbm, wtab_hbm, ptab_hbm, out_hbm,
              idx_w, idx_p, comb_v, ptab_v, sem_i, sem_w, sem_o):
    wid = lax.axis_index("s") * 2 + lax.axis_index("c")
    base_t = wid * NT

    def base_of(g):
        return jnp.minimum(base_t + g * C, N - C)

    def idx_copies(g, b):
        base = base_of(g)
        return [
            pltpu.make_async_copy(widx_hbm.at[pl.ds(base, C)],
                                  idx_w.at[b], sem_i),
            pltpu.make_async_copy(pidx_hbm.at[pl.ds(base, C)],
                                  idx_p.at[b], sem_i),
        ]

    def word_gather(b):
        return pltpu.make_async_copy(wtab_hbm.at[idx_w.at[b]],
                                     comb_v.at[b], sem_w)

    def out_copy(g, b):
        return pltpu.make_async_copy(comb_v.at[b],
                                     out_hbm.at[pl.ds(base_of(g), C)],
                                     sem_o)

    def pos_select(b):
        bvec = jnp.full((L,), b, jnp.int32)
        def grp(i, _):
            pv = idx_p[b, pl.ds(i * L, L)]
            rows = i * L + lax.iota(jnp.int32, L)
            for w in range(POS_DIM):
                cvec = jnp.full((L,), EMB + w, jnp.int32)
                val = plsc.load_gather(ptab_v, [pv, cvec])
                plsc.store_scatter(comb_v, [bvec, rows, cvec], val)
            return ()
        lax.fori_loop(0, C // L, grp, ())

    pltpu.sync_copy(ptab_hbm, ptab_v)

    def round_(k, first):
        gs = [k * NBUF + j for j in range(NBUF)]
        for j, g in enumerate(gs):
            for c in idx_copies(g, j):
                c.wait()
            if not first:
                out_copy(g - NBUF, j).wait()
            word_gather(j).start()
        for j, g in enumerate(gs):
            word_gather(j).wait()
            pos_select(j)
            out_copy(g, j).start()
            for c in idx_copies(g + NBUF, j):
                c.start()

    for j in range(NBUF):
        for c in idx_copies(j, j):
            c.start()
    round_(0, True)
    lax.fori_loop(1, NB // NBUF, lambda k, _: (round_(k, False), ())[1], ())
    for j in range(NBUF):
        out_copy(NB - NBUF + j, j).wait()
        for c in idx_copies(NB + j, j):
            c.wait()


@jax.jit
def _emb_call(widx, pidx, wtab_p, ptab_s):
    mesh = plsc.VectorSubcoreMesh(core_axis_name="c", subcore_axis_name="s")
    f = pl.kernel(
        _emb_body,
        out_type=jax.ShapeDtypeStruct((N, PAD_D), jnp.float32),
        mesh=mesh,
        compiler_params=pltpu.CompilerParams(needs_layout_passes=False),
        scratch_types=[
            pltpu.VMEM((NBUF, C), jnp.int32),
            pltpu.VMEM((NBUF, C), jnp.int32),
            pltpu.VMEM((NBUF, C, PAD_D), jnp.float32),
            pltpu.VMEM((64, PAD_D), jnp.float32),
            pltpu.SemaphoreType.DMA,
            pltpu.SemaphoreType.DMA,
            pltpu.SemaphoreType.DMA,
        ],
    )
    return f(widx, pidx, wtab_p, ptab_s)


def kernel(word_inputs, pos_inputs, word_table, pos_table):
    widx = word_inputs.astype(jnp.int32).reshape(N)
    pidx = pos_inputs.astype(jnp.int32).reshape(N)
    wtab_p = jnp.pad(word_table, ((0, 0), (0, PAD_D - EMB)))
    ptab_s = jnp.pad(pos_table, ((0, 0), (EMB, PAD_D - EMB - POS_DIM)))
    out = _emb_call(widx, pidx, wtab_p, ptab_s)
    return out[:, :OUT_D].reshape(B, S, OUT_D)
